# Initial kernel scaffold; baseline (speedup 1.0000x reference)
import jax
import jax.numpy as jnp
from jax.experimental import pallas as pl


def kernel(features, targets):
    raise NotImplementedError("write your pallas kernel here")


# speedup vs baseline: 2399.6208x; 2399.6208x over previous
import jax
import jax.numpy as jnp
from jax import lax
from jax.experimental import pallas as pl
from jax.experimental.pallas import tpu as pltpu

_K = 2
_MARGIN = 0.1
_ALPHA = 0.5
_BETA = 0.5
_NUM_CLASSES = 32
_NEG_INF = float('-inf')
_POS_INF = float('inf')


def _nt(a, b):
    return lax.dot_general(a, b, (((1,), (1,)), ((), ())),
                           preferred_element_type=jnp.float32)


def _loss_body(f_ref, tcol_ref, trow_ref, out_ref):
    f = f_ref[...]
    t_col = tcol_ref[...]
    t_row = trow_ref[...]
    n = f.shape[0]

    ff = f * f
    ones_row = jnp.ones((1, f.shape[1]), jnp.float32)
    sq_row = _nt(ones_row, ff)
    sq_col = jnp.sum(ff, axis=1, keepdims=True)

    g = _nt(f, f)
    dsq = sq_col + sq_row - 2.0 * g

    same = t_col == t_row
    masked = jnp.where(same, dsq, _NEG_INF)
    colmax = jnp.max(masked, axis=0, keepdims=True)

    labels = lax.broadcasted_iota(jnp.int32, (_NUM_CLASSES, 1), 0)
    onehot = (labels == t_row).astype(jnp.float32)
    counts_col = jnp.sum(onehot, axis=1, keepdims=True)

    cmask = labels == t_row
    cmax_dsq = jnp.max(jnp.where(cmask, colmax, _NEG_INF), axis=1,
                       keepdims=True)
    cmax = jnp.sqrt(jnp.clip(cmax_dsq, 1e-12, None))
    contrib = jnp.where(counts_col >= 2.0, 1.0 / cmax, 0.0)
    intra = jnp.sum(contrib)

    centers = lax.dot_general(onehot, f, (((1,), (0,)), ((), ())),
                              preferred_element_type=jnp.float32)
    centers = centers / jnp.maximum(counts_col, 1.0)
    cc = centers * centers
    csq_col = jnp.sum(cc, axis=1, keepdims=True)
    csq_row = _nt(jnp.ones((1, cc.shape[1]), jnp.float32), cc)
    gc = _nt(centers, centers)
    dc = jnp.sqrt(jnp.clip(csq_col + csq_row - 2.0 * gc, 1e-12, None))
    present_col = counts_col > 0.0
    counts_row = _nt(jnp.ones((1, n), jnp.float32), onehot)
    present_row = counts_row > 0.0
    valid = present_col & present_row & (dc > 0.0)
    min_inter = jnp.min(jnp.where(valid, dc, _POS_INF))

    loss = _ALPHA * (_MARGIN - min_inter) + _BETA * intra
    out_ref[0, 0] = loss


def kernel(features, targets):
    n = features.shape[0]
    t_col = targets.reshape(n, 1).astype(jnp.int32)
    t_row = targets.reshape(1, n).astype(jnp.int32)
    out = pl.pallas_call(
        _loss_body,
        out_shape=jax.ShapeDtypeStruct((1, 1), jnp.float32),
        out_specs=pl.BlockSpec(memory_space=pltpu.SMEM),
    )(features, t_col, t_row)
    return out[0, 0]
